# Initial kernel scaffold; baseline (speedup 1.0000x reference)
import jax
import jax.numpy as jnp
from jax.experimental import pallas as pl

_DIMS = [("e1",5,6,1,2),("n1",5,6,8,10),("e2",16,32,8,16),("n2",16,32,48,24),
         ("e3",56,24,48,24),("n3",56,24,48,24),("e4",48,13,48,13),("n4",48,13,26,8),
         ("e5",21,3,26,3),("n5",21,3,6,3)]


def _final_body(x_ref, o_ref):
    x = x_ref[...]
    s = jnp.sum(x, axis=0, keepdims=True)
    o_ref[...] = jnp.concatenate([jnp.broadcast_to(s, x.shape), x], axis=1)


def kernel(x, edge_index, edge_attr, params):
    N = x.shape[0]
    for nm, _ix, _ox, _ie, _oe in _DIMS:
        wx = params[nm + "_wx"]; bx = params[nm + "_bx"]
        we = params[nm + "_we"]; be = params[nm + "_be"]
        if nm.startswith("e"):
            y = x @ wx.T
            xs = y[edge_index[0]] + y[edge_index[1]] + bx
            ea = edge_attr @ we.T + be
            edge_attr = jax.nn.relu(jnp.concatenate([xs, ea], axis=1))
        else:
            agg = jax.ops.segment_sum(edge_attr, edge_index[0], num_segments=N)
            xs = x @ wx.T + bx
            ea = agg @ we.T + be
            x = jax.nn.relu(jnp.concatenate([xs, ea], axis=1))
    D = x.shape[1]
    return pl.pallas_call(
        _final_body,
        out_shape=jax.ShapeDtypeStruct((N, 2 * D), x.dtype),
    )(x)


# speedup vs baseline: 4.9230x; 4.9230x over previous
import functools

import jax
import jax.numpy as jnp
from jax import lax
from jax.experimental import pallas as pl
from jax.experimental.pallas import tpu as pltpu
from jax.experimental.pallas import tpu_sc as plsc

N = 2048
E = 65536
NC = 2
NS = 16
NW = NC * NS
L = 16
CE = E // NW
CH = 128
NPT = N // NS
LW = 128

_DIMS = [("e1",5,6,1,2),("n1",5,6,8,10),("e2",16,32,8,16),("n2",16,32,48,24),
         ("e3",56,24,48,24),("n3",56,24,48,24),("e4",48,13,48,13),("n4",48,13,26,8),
         ("e5",21,3,26,3),("n5",21,3,6,3)]
_EDGE = [d for d in _DIMS if d[0].startswith("e")]
_NODE = [d for d in _DIMS if d[0].startswith("n")]


def _pad8(n):
    return ((n + 7) // 8) * 8


def _sc_round(ox, oe, write_e):
    w = ox + oe
    oep = _pad8(oe)
    mesh = plsc.VectorSubcoreMesh(core_axis_name="c", subcore_axis_name="s")

    out_type = []
    if write_e:
        out_type.append(jax.ShapeDtypeStruct((E, LW), jnp.float32))
    out_type.append(jax.ShapeDtypeStruct((NC, N, LW), jnp.float32))

    NCH = CE // CH

    def body(y_hbm, src_hbm, dst_hbm, v_hbm, *refs):
        if write_e:
            e_hbm, agg_hbm = refs[:2]
            refs = refs[2:]
        else:
            agg_hbm = refs[0]
            refs = refs[1:]
        (y_v, srcb, dstb, vb, eb, agg_sh,
         ysem, in0, in1, in2, out0, out1) = refs
        aggv = eb[0]
        in_sems = [in0, in1, in2]
        out_sems = [out0, out1]
        cid = lax.axis_index("c")
        sid = lax.axis_index("s")
        wid = sid * NC + cid
        base0 = wid * CE

        ydesc = pltpu.async_copy(y_hbm, y_v, ysem)

        def issue_in(ch):
            base = base0 + ch * CH
            s = ch % 3
            return (
                pltpu.async_copy(src_hbm.at[pl.ds(base, CH)], srcb[s], in_sems[s]),
                pltpu.async_copy(dst_hbm.at[pl.ds(base, CH)], dstb[s], in_sems[s]),
                pltpu.async_copy(v_hbm.at[pl.ds(base, CH), pl.ds(0, oep)],
                                 vb[s], in_sems[s]),
            )

        pend_in = {0: issue_in(0)}

        zeros = jnp.zeros((L,), jnp.float32)

        def zeb(i, _):
            for c in range(LW // L):
                for e in range(2):
                    eb[e][i, pl.ds(c * L, L)] = zeros
            return 0

        lax.fori_loop(0, CH, zeb, 0)
        pltpu.sync_copy(aggv, agg_sh.at[pl.ds(sid * NPT, NPT)])

        plsc.subcore_barrier()
        ydesc.wait()

        iota = lax.iota(jnp.int32, L)
        pend_out = {}
        for ch in range(NCH):
            s = ch % 3
            e = ch % 2
            if ch + 1 < NCH:
                pend_in[ch + 1] = issue_in(ch + 1)
            for d in pend_in.pop(ch):
                d.wait()
            if ch - 2 in pend_out:
                for d in pend_out.pop(ch - 2):
                    d.wait()

            def group(g, _):
                g16 = g * L
                row16 = g16 + iota
                s16 = srcb[s][pl.ds(g16, L)]
                d16 = dstb[s][pl.ds(g16, L)]
                for c in range(ox):
                    c16 = jnp.full((L,), c, jnp.int32)
                    u = (plsc.load_gather(y_v, [s16, c16])
                         + plsc.load_gather(y_v, [d16, c16]))
                    plsc.store_scatter(eb[e], [row16, c16],
                                       jnp.maximum(u, 0.0))
                for c in range(oe):
                    c16 = jnp.full((L,), c, jnp.int32)
                    vv = plsc.load_gather(vb[s], [row16, c16])
                    plsc.store_scatter(eb[e], [row16, jnp.full((L,), ox + c, jnp.int32)],
                                       jnp.maximum(vv, 0.0))
                return 0

            lax.fori_loop(0, CH // L, group, 0)

            base = base0 + ch * CH
            outs = []
            if write_e:
                outs.append(pltpu.async_copy(eb[e], e_hbm.at[pl.ds(base, CH)],
                                             out_sems[e]))
            pltpu.sync_copy(eb[e], agg_sh.at[srcb[s]], add=True)
            pend_out[ch] = outs

        for ch in sorted(pend_out):
            for d in pend_out[ch]:
                d.wait()
        plsc.subcore_barrier()
        pltpu.sync_copy(agg_sh.at[pl.ds(sid * NPT, NPT)], aggv)
        pltpu.sync_copy(aggv, agg_hbm.at[cid].at[pl.ds(sid * NPT, NPT)])

    return pl.kernel(
        body,
        out_type=tuple(out_type),
        mesh=mesh,
        compiler_params=pltpu.CompilerParams(use_tc_tiling_on_sc=False,
                                             needs_layout_passes=False),
        scratch_types=[
            pltpu.VMEM((N, ox), jnp.float32),
            [pltpu.VMEM((CH,), jnp.int32)] * 3,
            [pltpu.VMEM((CH,), jnp.int32)] * 3,
            [pltpu.VMEM((CH, oep), jnp.float32)] * 3,
            [pltpu.VMEM((CH, LW), jnp.float32)] * 2,
            pltpu.VMEM_SHARED((N, LW), jnp.float32),
            pltpu.SemaphoreType.DMA,
            pltpu.SemaphoreType.DMA,
            pltpu.SemaphoreType.DMA,
            pltpu.SemaphoreType.DMA,
            pltpu.SemaphoreType.DMA,
            pltpu.SemaphoreType.DMA,
        ],
    )


_BE = 8192


def _tc_init(x, ea, wxT, bx2, we_row, be_row):
    ox = wxT.shape[1]

    def ybody(x_ref, w_ref, b_ref, y_ref):
        y_ref[...] = jnp.dot(x_ref[...], w_ref[...],
                             preferred_element_type=jnp.float32) + b_ref[...]

    y = pl.pallas_call(
        ybody,
        out_shape=jax.ShapeDtypeStruct((N, ox), jnp.float32),
    )(x, wxT, bx2)

    def vbody(e_ref, w_ref, b_ref, v_ref):
        v_ref[...] = e_ref[...] * w_ref[...] + b_ref[...]

    grid = (E // _BE,)
    v = pl.pallas_call(
        vbody,
        grid=grid,
        in_specs=[pl.BlockSpec((_BE, 1), lambda i: (i, 0)),
                  pl.BlockSpec(we_row.shape, lambda i: (0, 0)),
                  pl.BlockSpec(be_row.shape, lambda i: (0, 0))],
        out_specs=pl.BlockSpec((_BE, LW), lambda i: (i, 0)),
        out_shape=jax.ShapeDtypeStruct((E, LW), jnp.float32),
    )(ea, we_row, be_row)
    return y, v


def _tc_node(x, aggp, e_next, w_cur, wxnT, bxn, wenT, ben,
             wxeT, bxe2, weeT, bee):
    nox = wxnT.shape[1]
    noe = wenT.shape[1]
    ox_n = wxeT.shape[1]
    oe_n = weeT.shape[1]
    ix1 = nox + noe

    def body(e_ref, x_ref, agg_ref, wxn_ref, bxn_ref, wen_ref, ben_ref,
             wxe_ref, bxe_ref, wee_ref, bee_ref,
             v_ref, x1_ref, y_ref):
        vv = jnp.dot(e_ref[:, :w_cur], wee_ref[...],
                     preferred_element_type=jnp.float32) + bee_ref[...]
        v_ref[...] = jnp.concatenate(
            [vv, jnp.zeros((vv.shape[0], LW - oe_n), jnp.float32)], axis=1)

        @pl.when(pl.program_id(0) == 0)
        def _():
            agg = agg_ref[0, :, :w_cur] + agg_ref[1, :, :w_cur]
            xs = jnp.dot(x_ref[...], wxn_ref[...],
                         preferred_element_type=jnp.float32) + bxn_ref[...]
            ea = jnp.dot(agg, wen_ref[...],
                         preferred_element_type=jnp.float32) + ben_ref[...]
            x1 = jnp.maximum(jnp.concatenate([xs, ea], axis=1), 0.0)
            x1_ref[...] = x1
            y_ref[...] = jnp.dot(x1, wxe_ref[...],
                                 preferred_element_type=jnp.float32) + bxe_ref[...]

    grid = (E // _BE,)
    full = lambda s: pl.BlockSpec(s, lambda i: tuple(0 for _ in s))
    return pl.pallas_call(
        body,
        grid=grid,
        in_specs=[pl.BlockSpec((_BE, LW), lambda i: (i, 0)),
                  full(x.shape), full(aggp.shape),
                  full(wxnT.shape), full(bxn.shape),
                  full(wenT.shape), full(ben.shape),
                  full(wxeT.shape), full(bxe2.shape),
                  full(weeT.shape), full(bee.shape)],
        out_specs=[pl.BlockSpec((_BE, LW), lambda i: (i, 0)),
                   full((N, ix1)), full((N, ox_n))],
        out_shape=[jax.ShapeDtypeStruct((E, LW), jnp.float32),
                   jax.ShapeDtypeStruct((N, ix1), jnp.float32),
                   jax.ShapeDtypeStruct((N, ox_n), jnp.float32)],
    )(e_next, x, aggp, wxnT, bxn, wenT, ben, wxeT, bxe2, weeT, bee)


def _tc_final(x, aggp, w_cur, wxnT, bxn, wenT, ben):
    nox = wxnT.shape[1]
    noe = wenT.shape[1]
    D = nox + noe

    def body(x_ref, agg_ref, wxn_ref, bxn_ref, wen_ref, ben_ref, o_ref):
        agg = agg_ref[0, :, :w_cur] + agg_ref[1, :, :w_cur]
        xs = jnp.dot(x_ref[...], wxn_ref[...],
                     preferred_element_type=jnp.float32) + bxn_ref[...]
        ea = jnp.dot(agg, wen_ref[...],
                     preferred_element_type=jnp.float32) + ben_ref[...]
        x1 = jnp.maximum(jnp.concatenate([xs, ea], axis=1), 0.0)
        s = jnp.sum(x1, axis=0, keepdims=True)
        o_ref[...] = jnp.concatenate(
            [jnp.broadcast_to(s, (N, D)), x1], axis=1)

    return pl.pallas_call(
        body,
        out_shape=jax.ShapeDtypeStruct((N, 2 * D), jnp.float32),
    )(x, aggp, wxnT, bxn, wenT, ben)


def _padT(wm, rows_p):
    o, i = wm.shape
    wT = wm.T
    if rows_p > i:
        wT = jnp.concatenate(
            [wT, jnp.zeros((rows_p - i, o), jnp.float32)], axis=0)
    return wT


def kernel(x, edge_index, edge_attr, params):
    src = edge_index[0].astype(jnp.int32)
    dst = edge_index[1].astype(jnp.int32)

    p = params
    row = lambda b: b.reshape(1, -1)

    def rowpad(b, n):
        b = b.reshape(1, -1)
        return jnp.concatenate(
            [b, jnp.zeros((1, n - b.shape[1]), jnp.float32)], axis=1)

    we1_row = rowpad(p["e1_we"][:, 0], LW)
    be1_row = rowpad(p["e1_be"], LW)
    y, v = _tc_init(x, edge_attr,
                    p["e1_wx"].T, row(p["e1_bx"] * 0.5),
                    we1_row, be1_row)

    for k in range(5):
        enm, _ix, ox, _ie, oe = _EDGE[k]
        nnm, nix, nox, nie, noe = _NODE[k]
        w_cur = ox + oe
        write_e = k < 4
        outs = _sc_round(ox, oe, write_e)(y, src, dst, v)
        if write_e:
            e_next, aggp = outs
        else:
            aggp, = outs

        if k < 4:
            e2nm = _EDGE[k + 1][0]
            oe2 = _EDGE[k + 1][4]
            v, x, y = _tc_node(
                x, aggp, e_next, w_cur,
                p[nnm + "_wx"].T, row(p[nnm + "_bx"]),
                _padT(p[nnm + "_we"], w_cur), row(p[nnm + "_be"]),
                p[e2nm + "_wx"].T, row(p[e2nm + "_bx"] * 0.5),
                _padT(p[e2nm + "_we"], w_cur), row(p[e2nm + "_be"]))
        else:
            out = _tc_final(
                x, aggp, w_cur,
                p[nnm + "_wx"].T, row(p[nnm + "_bx"]),
                _padT(p[nnm + "_we"], w_cur), row(p[nnm + "_be"]))
    return out
